# baseline (speedup 1.0000x reference)
import jax
import jax.numpy as jnp
from jax.experimental import pallas as pl
from jax.experimental.pallas import tpu as pltpu

_E = 8192
_H = _E // 2
_D = 32
_N = 16384
_TR = 128
_STEPS = _N // _TR
_COMMIT = 0.25


def _vq_kernel(x_ref, x2_ref, w_ref, enc_ref, q_ref, cnt_ref, loss_ref, perp_ref):
    x = x_ref[...]
    w = w_ref[...]
    x2 = x2_ref[...]
    w2 = jnp.sum(w * w, axis=1, keepdims=True)
    xb = x.astype(jnp.bfloat16)
    mm = jax.lax.dot_general(xb, w, (((1,), (1,)), ((), ())),
                             preferred_element_type=jnp.float32)
    d = (x2 + w2.T) - 2.0 * mm

    iota = jax.lax.broadcasted_iota(jnp.int32, (_TR, _H), 1)
    d0 = d[:, :_H]
    d1 = d[:, _H:]
    m0 = jnp.min(d0, axis=1, keepdims=True)
    i0 = jnp.min(jnp.where(d0 == m0, iota, _E), axis=1, keepdims=True)
    m1 = jnp.min(d1, axis=1, keepdims=True)
    i1 = jnp.min(jnp.where(d1 == m1, iota + _H, _E), axis=1, keepdims=True)
    m0b = m0.astype(jnp.bfloat16).astype(jnp.float32)
    idx = jnp.where(m1 < m0b, i1, i0)

    iota_full = jax.lax.broadcasted_iota(jnp.int32, (_TR, _E), 1)
    enc = (iota_full == idx).astype(jnp.float32)
    enc_ref[...] = enc
    q = jax.lax.dot_general(enc, w, (((1,), (0,)), ((), ())))
    q_ref[...] = x + (q - x)

    step = pl.program_id(0)
    cnt = jnp.sum(enc, axis=0, keepdims=True)
    sse = jnp.sum((q - x) ** 2, axis=(0, 1), keepdims=True)

    @pl.when(step == 0)
    def _init():
        cnt_ref[...] = jnp.zeros_like(cnt_ref)
        loss_ref[...] = jnp.zeros_like(loss_ref)
        perp_ref[...] = jnp.zeros_like(perp_ref)

    cnt_ref[...] += cnt
    loss_ref[...] += sse

    @pl.when(step == _STEPS - 1)
    def _finalize():
        m = loss_ref[...] * (1.0 / (_N * _D))
        loss_ref[...] = m + _COMMIT * m
        p = cnt_ref[...] * (1.0 / _N)
        s = jnp.sum(p * jnp.log(p + 1e-10), axis=(0, 1), keepdims=True)
        perp_ref[...] = jnp.exp(-s)


def kernel(inputs, W):
    flat = inputs.reshape(-1, _D)
    x2 = jnp.sum(inputs ** 2, axis=2).reshape(-1, 1)
    enc, q, _cnt, loss, perp = pl.pallas_call(
        _vq_kernel,
        grid=(_STEPS,),
        in_specs=[
            pl.BlockSpec((_TR, _D), lambda i: (i, 0)),
            pl.BlockSpec((_TR, 1), lambda i: (i, 0)),
            pl.BlockSpec((_E, _D), lambda i: (0, 0)),
        ],
        out_specs=[
            pl.BlockSpec((_TR, _E), lambda i: (i, 0)),
            pl.BlockSpec((_TR, _D), lambda i: (i, 0)),
            pl.BlockSpec((1, _E), lambda i: (0, 0)),
            pl.BlockSpec((1, 1), lambda i: (0, 0)),
            pl.BlockSpec((1, 1), lambda i: (0, 0)),
        ],
        out_shape=[
            jax.ShapeDtypeStruct((_N, _E), jnp.float32),
            jax.ShapeDtypeStruct((_N, _D), jnp.float32),
            jax.ShapeDtypeStruct((1, _E), jnp.float32),
            jax.ShapeDtypeStruct((1, 1), jnp.float32),
            jax.ShapeDtypeStruct((1, 1), jnp.float32),
        ],
        compiler_params=pltpu.CompilerParams(
            dimension_semantics=("arbitrary",),
        ),
    )(flat, x2, W)
    return (loss.reshape(()), q.reshape(inputs.shape), perp.reshape(()), enc)


# speedup vs baseline: 1.0864x; 1.0864x over previous
import jax
import jax.numpy as jnp
from jax.experimental import pallas as pl
from jax.experimental.pallas import tpu as pltpu

_E = 8192
_H = _E // 2
_D = 32
_N = 16384
_TR = 256
_STEPS = _N // _TR
_COMMIT = 0.25


def _vq_kernel(x_ref, x2_ref, w_ref, enc_ref, q_ref, cnt_ref, loss_ref, perp_ref):
    x = x_ref[...]
    w = w_ref[...]
    x2 = x2_ref[...]
    w2 = jnp.sum(w * w, axis=1, keepdims=True)
    xb = x.astype(jnp.bfloat16)
    mm = jax.lax.dot_general(xb, w, (((1,), (1,)), ((), ())),
                             preferred_element_type=jnp.float32)
    d = (x2 + w2.T) - 2.0 * mm

    iota = jax.lax.broadcasted_iota(jnp.int32, (_TR, _H), 1)
    d0 = d[:, :_H]
    d1 = d[:, _H:]
    m0 = jnp.min(d0, axis=1, keepdims=True)
    i0 = jnp.min(jnp.where(d0 == m0, iota, _E), axis=1, keepdims=True)
    m1 = jnp.min(d1, axis=1, keepdims=True)
    i1 = jnp.min(jnp.where(d1 == m1, iota + _H, _E), axis=1, keepdims=True)
    m0b = m0.astype(jnp.bfloat16).astype(jnp.float32)
    idx = jnp.where(m1 < m0b, i1, i0)

    iota_full = jax.lax.broadcasted_iota(jnp.int32, (_TR, _E), 1)
    enc = (iota_full == idx).astype(jnp.float32)
    enc_ref[...] = enc
    q = jax.lax.dot_general(enc, w, (((1,), (0,)), ((), ())))
    q_ref[...] = x + (q - x)

    step = pl.program_id(0)
    ones = jnp.ones((1, _TR), jnp.float32)
    cnt = jax.lax.dot_general(ones, enc, (((1,), (0,)), ((), ())))
    sse = jnp.sum((q - x) ** 2, axis=(0, 1), keepdims=True)

    @pl.when(step == 0)
    def _init():
        cnt_ref[...] = jnp.zeros_like(cnt_ref)
        loss_ref[...] = jnp.zeros_like(loss_ref)
        perp_ref[...] = jnp.zeros_like(perp_ref)

    cnt_ref[...] += cnt
    loss_ref[...] += sse

    @pl.when(step == _STEPS - 1)
    def _finalize():
        m = loss_ref[...] * (1.0 / (_N * _D))
        loss_ref[...] = m + _COMMIT * m
        p = cnt_ref[...] * (1.0 / _N)
        s = jnp.sum(p * jnp.log(p + 1e-10), axis=(0, 1), keepdims=True)
        perp_ref[...] = jnp.exp(-s)


def kernel(inputs, W):
    flat = inputs.reshape(-1, _D)
    x2 = jnp.sum(inputs ** 2, axis=2).reshape(-1, 1)
    enc, q, _cnt, loss, perp = pl.pallas_call(
        _vq_kernel,
        grid=(_STEPS,),
        in_specs=[
            pl.BlockSpec((_TR, _D), lambda i: (i, 0)),
            pl.BlockSpec((_TR, 1), lambda i: (i, 0)),
            pl.BlockSpec((_E, _D), lambda i: (0, 0)),
        ],
        out_specs=[
            pl.BlockSpec((_TR, _E), lambda i: (i, 0)),
            pl.BlockSpec((_TR, _D), lambda i: (i, 0)),
            pl.BlockSpec((1, _E), lambda i: (0, 0)),
            pl.BlockSpec((1, 1), lambda i: (0, 0)),
            pl.BlockSpec((1, 1), lambda i: (0, 0)),
        ],
        out_shape=[
            jax.ShapeDtypeStruct((_N, _E), jnp.float32),
            jax.ShapeDtypeStruct((_N, _D), jnp.float32),
            jax.ShapeDtypeStruct((1, _E), jnp.float32),
            jax.ShapeDtypeStruct((1, 1), jnp.float32),
            jax.ShapeDtypeStruct((1, 1), jnp.float32),
        ],
        compiler_params=pltpu.CompilerParams(
            dimension_semantics=("arbitrary",),
        ),
    )(flat, x2, W)
    return (loss.reshape(()), q.reshape(inputs.shape), perp.reshape(()), enc)


# speedup vs baseline: 1.1982x; 1.1029x over previous
import jax
import jax.numpy as jnp
from jax.experimental import pallas as pl
from jax.experimental.pallas import tpu as pltpu

_E = 8192
_H = _E // 2
_D = 32
_N = 16384
_TR = 256
_STEPS = _N // _TR
_COMMIT = 0.25


def _vq_kernel(x_ref, x2_ref, w_ref, w2x_ref, it_ref, enc_ref, q_ref, cnt_ref,
               loss_ref, perp_ref):
    x = x_ref[...]
    w = w_ref[...]
    w2x = w2x_ref[...]
    it = it_ref[...]
    x2 = x2_ref[...]
    w2 = jnp.sum(w * w, axis=1, keepdims=True)
    xb = x.astype(jnp.bfloat16)
    mm2 = jax.lax.dot_general(xb, w2x, (((1,), (1,)), ((), ())),
                              preferred_element_type=jnp.float32)
    d = (x2 + w2.T) - mm2

    d0 = d[:, :_H]
    d1 = d[:, _H:]
    it0 = it[:, :_H]
    it1 = it[:, _H:]
    m0 = jnp.min(d0, axis=1, keepdims=True)
    i0 = jnp.min(jnp.where(d0 == m0, it0, float(_E)), axis=1, keepdims=True)
    m1 = jnp.min(d1, axis=1, keepdims=True)
    i1 = jnp.min(jnp.where(d1 == m1, it1, float(_E)), axis=1, keepdims=True)
    m0b = m0.astype(jnp.bfloat16).astype(jnp.float32)
    idx = jnp.where(m1 < m0b, i1, i0)

    enc = (it == idx).astype(jnp.float32)
    enc_ref[...] = enc
    q = jax.lax.dot_general(enc, w, (((1,), (0,)), ((), ())))
    q_ref[...] = x + (q - x)

    step = pl.program_id(0)
    ones = jnp.ones((1, _TR), jnp.float32)
    cnt = jax.lax.dot_general(ones, enc, (((1,), (0,)), ((), ())))
    sse = jnp.sum((q - x) ** 2, axis=(0, 1), keepdims=True)

    @pl.when(step == 0)
    def _init():
        cnt_ref[...] = jnp.zeros_like(cnt_ref)
        loss_ref[...] = jnp.zeros_like(loss_ref)
        perp_ref[...] = jnp.zeros_like(perp_ref)

    cnt_ref[...] += cnt
    loss_ref[...] += sse

    @pl.when(step == _STEPS - 1)
    def _finalize():
        m = loss_ref[...] * (1.0 / (_N * _D))
        loss_ref[...] = m + _COMMIT * m
        p = cnt_ref[...] * (1.0 / _N)
        s = jnp.sum(p * jnp.log(p + 1e-10), axis=(0, 1), keepdims=True)
        perp_ref[...] = jnp.exp(-s)


def kernel(inputs, W):
    flat = inputs.reshape(-1, _D)
    x2 = jnp.sum(inputs ** 2, axis=2).reshape(-1, 1)
    w2x = 2.0 * W
    it = jnp.arange(_E, dtype=jnp.float32).reshape(1, _E)
    enc, q, _cnt, loss, perp = pl.pallas_call(
        _vq_kernel,
        grid=(_STEPS,),
        in_specs=[
            pl.BlockSpec((_TR, _D), lambda i: (i, 0)),
            pl.BlockSpec((_TR, 1), lambda i: (i, 0)),
            pl.BlockSpec((_E, _D), lambda i: (0, 0)),
            pl.BlockSpec((_E, _D), lambda i: (0, 0)),
            pl.BlockSpec((1, _E), lambda i: (0, 0)),
        ],
        out_specs=[
            pl.BlockSpec((_TR, _E), lambda i: (i, 0)),
            pl.BlockSpec((_TR, _D), lambda i: (i, 0)),
            pl.BlockSpec((1, _E), lambda i: (0, 0)),
            pl.BlockSpec((1, 1), lambda i: (0, 0)),
            pl.BlockSpec((1, 1), lambda i: (0, 0)),
        ],
        out_shape=[
            jax.ShapeDtypeStruct((_N, _E), jnp.float32),
            jax.ShapeDtypeStruct((_N, _D), jnp.float32),
            jax.ShapeDtypeStruct((1, _E), jnp.float32),
            jax.ShapeDtypeStruct((1, 1), jnp.float32),
            jax.ShapeDtypeStruct((1, 1), jnp.float32),
        ],
        compiler_params=pltpu.CompilerParams(
            dimension_semantics=("arbitrary",),
        ),
    )(flat, x2, W, w2x, it)
    return (loss.reshape(()), q.reshape(inputs.shape), perp.reshape(()), enc)


# speedup vs baseline: 1.4065x; 1.1739x over previous
import jax
import jax.numpy as jnp
from jax.experimental import pallas as pl
from jax.experimental.pallas import tpu as pltpu

_E = 8192
_H = _E // 2
_D = 32
_N = 16384
_TR = 512
_STEPS = _N // _TR
_COMMIT = 0.25


def _vq_kernel(x_ref, x2_ref, w2x_ref, it_ref, enc_ref, q_ref, cnt_ref,
               loss_ref, perp_ref):
    x = x_ref[...]
    w2x = w2x_ref[...]
    w = w2x * 0.5
    it = it_ref[...]
    x2 = x2_ref[...]
    w2 = jnp.sum(w * w, axis=1, keepdims=True)
    xb = x.astype(jnp.bfloat16)
    mm2 = jax.lax.dot_general(xb, w2x, (((1,), (1,)), ((), ())),
                              preferred_element_type=jnp.float32)
    d = (x2 + w2.T) - mm2

    d0 = d[:, :_H]
    d1 = d[:, _H:]
    it0 = it[:, :_H]
    it1 = it[:, _H:]
    m0 = jnp.min(d0, axis=1, keepdims=True)
    i0 = jnp.min(jnp.where(d0 == m0, it0, float(_E)), axis=1, keepdims=True)
    m1 = jnp.min(d1, axis=1, keepdims=True)
    i1 = jnp.min(jnp.where(d1 == m1, it1, float(_E)), axis=1, keepdims=True)
    m0b = m0.astype(jnp.bfloat16).astype(jnp.float32)
    win1 = m1 < m0b
    idx = jnp.where(win1, i1, i0)
    dsel = jnp.where(win1, m1, m0)

    enc = (it == idx).astype(jnp.float32)
    enc_ref[...] = enc
    q = jax.lax.dot_general(enc, w, (((1,), (0,)), ((), ())))
    q_ref[...] = x + (q - x)

    step = pl.program_id(0)
    ones = jnp.ones((1, _TR), jnp.float32)
    cnt = jax.lax.dot_general(ones, enc, (((1,), (0,)), ((), ())))
    sse = jnp.sum(dsel, axis=(0, 1), keepdims=True)

    @pl.when(step == 0)
    def _init():
        cnt_ref[...] = jnp.zeros_like(cnt_ref)
        loss_ref[...] = jnp.zeros_like(loss_ref)
        perp_ref[...] = jnp.zeros_like(perp_ref)

    cnt_ref[...] += cnt
    loss_ref[...] += sse

    @pl.when(step == _STEPS - 1)
    def _finalize():
        m = loss_ref[...] * (1.0 / (_N * _D))
        loss_ref[...] = m + _COMMIT * m
        p = cnt_ref[...] * (1.0 / _N)
        s = jnp.sum(p * jnp.log(p + 1e-10), axis=(0, 1), keepdims=True)
        perp_ref[...] = jnp.exp(-s)


def kernel(inputs, W):
    flat = inputs.reshape(-1, _D)
    x2 = jnp.sum(inputs ** 2, axis=2).reshape(-1, 1)
    w2x = 2.0 * W
    it = jnp.arange(_E, dtype=jnp.float32).reshape(1, _E)
    enc, q, _cnt, loss, perp = pl.pallas_call(
        _vq_kernel,
        grid=(_STEPS,),
        in_specs=[
            pl.BlockSpec((_TR, _D), lambda i: (i, 0)),
            pl.BlockSpec((_TR, 1), lambda i: (i, 0)),
            pl.BlockSpec((_E, _D), lambda i: (0, 0)),
            pl.BlockSpec((1, _E), lambda i: (0, 0)),
        ],
        out_specs=[
            pl.BlockSpec((_TR, _E), lambda i: (i, 0)),
            pl.BlockSpec((_TR, _D), lambda i: (i, 0)),
            pl.BlockSpec((1, _E), lambda i: (0, 0)),
            pl.BlockSpec((1, 1), lambda i: (0, 0)),
            pl.BlockSpec((1, 1), lambda i: (0, 0)),
        ],
        out_shape=[
            jax.ShapeDtypeStruct((_N, _E), jnp.float32),
            jax.ShapeDtypeStruct((_N, _D), jnp.float32),
            jax.ShapeDtypeStruct((1, _E), jnp.float32),
            jax.ShapeDtypeStruct((1, 1), jnp.float32),
            jax.ShapeDtypeStruct((1, 1), jnp.float32),
        ],
        compiler_params=pltpu.CompilerParams(
            dimension_semantics=("arbitrary",),
        ),
    )(flat, x2, w2x, it)
    return (loss.reshape(()), q.reshape(inputs.shape), perp.reshape(()), enc)


# speedup vs baseline: 1.7482x; 1.2429x over previous
import functools

import jax
import jax.numpy as jnp
from jax import lax
from jax.experimental import pallas as pl
from jax.experimental.pallas import tpu as pltpu
from jax.experimental.pallas import tpu_sc as plsc

_E = 8192
_H = _E // 2
_D = 32
_N = 16384
_TR = 512
_STEPS = _N // _TR
_COMMIT = 0.25


def _vq_kernel(x_ref, x2_ref, w2x_ref, it_ref, enc_ref, idx_ref, cnt_ref,
               loss_ref, perp_ref):
    x = x_ref[...]
    w2x = w2x_ref[...]
    it = it_ref[...]
    x2 = x2_ref[...]
    w = w2x * 0.5
    w2 = jnp.sum(w * w, axis=1, keepdims=True)
    xb = x.astype(jnp.bfloat16)
    mm2 = jax.lax.dot_general(xb, w2x, (((1,), (1,)), ((), ())),
                              preferred_element_type=jnp.float32)
    d = (x2 + w2.T) - mm2

    d0 = d[:, :_H]
    d1 = d[:, _H:]
    it0 = it[:, :_H]
    it1 = it[:, _H:]
    m0 = jnp.min(d0, axis=1, keepdims=True)
    i0 = jnp.min(jnp.where(d0 == m0, it0, float(_E)), axis=1, keepdims=True)
    m1 = jnp.min(d1, axis=1, keepdims=True)
    i1 = jnp.min(jnp.where(d1 == m1, it1, float(_E)), axis=1, keepdims=True)
    m0b = m0.astype(jnp.bfloat16).astype(jnp.float32)
    win1 = m1 < m0b
    idx = jnp.where(win1, i1, i0)
    dsel = jnp.where(win1, m1, m0)
    idx_ref[...] = idx.astype(jnp.int32)

    enc = (it == idx).astype(jnp.float32)
    enc_ref[...] = enc

    step = pl.program_id(0)
    ones = jnp.ones((1, _TR), jnp.float32)
    cnt = jax.lax.dot_general(ones, enc, (((1,), (0,)), ((), ())))
    sse = jnp.sum(dsel, axis=(0, 1), keepdims=True)

    @pl.when(step == 0)
    def _init():
        cnt_ref[...] = jnp.zeros_like(cnt_ref)
        loss_ref[...] = jnp.zeros_like(loss_ref)
        perp_ref[...] = jnp.zeros_like(perp_ref)

    cnt_ref[...] += cnt
    loss_ref[...] += sse

    @pl.when(step == _STEPS - 1)
    def _finalize():
        m = loss_ref[...] * (1.0 / (_N * _D))
        loss_ref[...] = m + _COMMIT * m
        p = cnt_ref[...] * (1.0 / _N)
        s = jnp.sum(p * jnp.log(p + 1e-10), axis=(0, 1), keepdims=True)
        perp_ref[...] = jnp.exp(-s)


_DP = 128


def _sc_gather(Wp, idx):
    info = plsc.get_sparse_core_info()
    nw = info.num_cores * info.num_subcores
    b_per_w = _N // nw
    mesh = plsc.VectorSubcoreMesh(core_axis_name="c", subcore_axis_name="s")

    @functools.partial(
        pl.kernel, mesh=mesh,
        out_type=jax.ShapeDtypeStruct((_N, _DP), jnp.float32),
        scratch_types=[
            pltpu.VMEM((b_per_w,), jnp.int32),
            pltpu.VMEM((b_per_w, _DP), jnp.float32),
            pltpu.SemaphoreType.DMA,
        ],
    )
    def k(table_hbm, idx_hbm, out_hbm, idx_v, rows_v, sem):
        wid = lax.axis_index("s") * info.num_cores + lax.axis_index("c")
        base = wid * b_per_w
        pltpu.sync_copy(idx_hbm.at[pl.ds(base, b_per_w)], idx_v)
        pltpu.async_copy(table_hbm.at[idx_v], rows_v, sem).wait()
        pltpu.sync_copy(rows_v, out_hbm.at[pl.ds(base, b_per_w)])

    return k(Wp, idx)


def kernel(inputs, W):
    flat = inputs.reshape(-1, _D)
    x2 = jnp.sum(inputs ** 2, axis=2).reshape(-1, 1)
    w2x = 2.0 * W
    it = jnp.arange(_E, dtype=jnp.float32).reshape(1, _E)
    enc, idx, _cnt, loss, perp = pl.pallas_call(
        _vq_kernel,
        grid=(_STEPS,),
        in_specs=[
            pl.BlockSpec((_TR, _D), lambda i: (i, 0)),
            pl.BlockSpec((_TR, 1), lambda i: (i, 0)),
            pl.BlockSpec((_E, _D), lambda i: (0, 0)),
            pl.BlockSpec((1, _E), lambda i: (0, 0)),
        ],
        out_specs=[
            pl.BlockSpec((_TR, _E), lambda i: (i, 0)),
            pl.BlockSpec((_TR, 1), lambda i: (i, 0)),
            pl.BlockSpec((1, _E), lambda i: (0, 0)),
            pl.BlockSpec((1, 1), lambda i: (0, 0)),
            pl.BlockSpec((1, 1), lambda i: (0, 0)),
        ],
        out_shape=[
            jax.ShapeDtypeStruct((_N, _E), jnp.float32),
            jax.ShapeDtypeStruct((_N, 1), jnp.int32),
            jax.ShapeDtypeStruct((1, _E), jnp.float32),
            jax.ShapeDtypeStruct((1, 1), jnp.float32),
            jax.ShapeDtypeStruct((1, 1), jnp.float32),
        ],
        compiler_params=pltpu.CompilerParams(
            dimension_semantics=("arbitrary",),
        ),
    )(flat, x2, w2x, it)
    Wp = jnp.pad(W, ((0, 0), (0, _DP - _D)))
    q = _sc_gather(Wp, idx.reshape(-1))[:, :_D]
    return (loss.reshape(()), q.reshape(inputs.shape), perp.reshape(()), enc)
